# baseline (speedup 1.0000x reference)
import functools
import math

import jax
import jax.numpy as jnp
from jax import lax
from jax.experimental import pallas as pl
from jax.experimental.pallas import tpu as pltpu
from jax.experimental.pallas import tpu_sc as plsc

_NW = 32
_L = 16
_TC = 128


def _relayout_body(n_blocks, tail_n, scale, tabT_hbm, tail_hbm, out_hbm,
                   ibuf, obuf, tailbuf, gsem0, gsem1, osem0, osem1):
  c = lax.axis_index("c")
  s = lax.axis_index("s")
  wid = s * 2 + c
  d = tabT_hbm.shape[0]
  n_iter = (n_blocks + _NW - 1) // _NW

  ibufs = (ibuf.at[0], ibuf.at[1])
  obufs = (obuf.at[0], obuf.at[1])
  gsems = (gsem0, gsem1)
  osems = (osem0, osem1)

  iota = lax.iota(jnp.int32, _L)
  zeros = jnp.full((_L,), 0, jnp.int32)

  def src_ref(ct):
    return tabT_hbm.at[:, pl.ds(ct * _TC, _TC)]

  def dst_ref(ct):
    return out_hbm.at[pl.ds(ct * (_TC // 2), _TC // 2)]

  def start_read(ct, k):
    pltpu.async_copy(src_ref(ct), ibufs[k], gsems[k])

  def wait_read(ct, k):
    pltpu.make_async_copy(src_ref(ct), ibufs[k], gsems[k]).wait()

  def start_write(ct, k):
    pltpu.async_copy(obufs[k], dst_ref(ct), osems[k])

  def wait_write(ct, k):
    pltpu.make_async_copy(obufs[k], dst_ref(ct), osems[k]).wait()

  def transpose_block(k):
    src = ibufs[k]
    dst = obufs[k]

    def row_body(i, carry):
      for jb in range(2 * d // _L):
        col = zeros + (2 * i + (jb * _L) // d)
        vals = plsc.load_gather(src, [iota + (jb * _L) % d, col])
        dst[i, pl.ds(jb * _L, _L)] = vals * scale
      return carry

    lax.fori_loop(0, _TC // 2, row_body, 0, unroll=2)

  def ct_of(i):
    return wid + _NW * i

  rd0 = ct_of(0) < n_blocks

  @pl.when(rd0)
  def _():
    start_read(ct_of(0), 0)

  @pl.loop(0, n_iter, step=2)
  def _(ii):
    for kk in range(2):
      i = ii + kk
      nk = 1 - kk
      @pl.when(ct_of(i + 1) < n_blocks)
      def _():
        @pl.when(i >= 1)
        def _():
          wait_write(ct_of(i - 1), nk)

        start_read(ct_of(i + 1), nk)

      @pl.when(ct_of(i) < n_blocks)
      def _():
        wait_read(ct_of(i), kk)
        transpose_block(kk)
        start_write(ct_of(i), kk)

  n_mine = (n_blocks - wid + _NW - 1) // _NW
  for kk in range(2):
    i_k = n_mine - 1 - lax.rem(n_mine - 1 - kk + 2, 2)

    @pl.when(i_k >= 0)
    def _():
      wait_write(ct_of(i_k), kk)

  if tail_n:
    @pl.when(wid == 0)
    def _():
      pltpu.sync_copy(tail_hbm, tailbuf)

      def trow(r, carry):
        for jb in range(2 * d // _L):
          v0 = 2 * r + (jb * _L) // d
          e0 = (jb * _L) % d
          vals = plsc.load_gather(tailbuf, [v0 * d + e0 + iota])
          obufs[0][r, pl.ds(jb * _L, _L)] = vals * scale
        return carry

      lax.fori_loop(0, tail_n // 2, trow, 0)
      pltpu.sync_copy(
          obufs[0].at[pl.ds(0, tail_n // 2)],
          out_hbm.at[pl.ds(n_blocks * (_TC // 2), tail_n // 2)])


def _gather_body(n_s, d, idx_hbm, tab_hbm, out_hbm, idx_v, rbuf, qbuf,
                 gsem0, gsem1, osem0, osem1):
  c = lax.axis_index("c")
  s = lax.axis_index("s")
  wid = s * 2 + c

  pltpu.sync_copy(idx_hbm.at[:, wid], idx_v)

  rbufs = (rbuf.at[0], rbuf.at[1])
  qbufs = (qbuf.at[0], qbuf.at[1])
  gsems = (gsem0, gsem1)
  osems = (osem0, osem1)

  iota = lax.iota(jnp.int32, _L)

  def start_gather(g, k):
    pltpu.async_copy(tab_hbm.at[idx_v.at[g]], rbufs[k], gsems[k])

  def wait_gather(g, k):
    pltpu.make_async_copy(tab_hbm.at[idx_v.at[g]], rbufs[k], gsems[k]).wait()

  def out_ref(g):
    return out_hbm.at[g, :, wid]

  def start_out(g, k):
    pltpu.async_copy(qbufs[k], out_ref(g), osems[k])

  def wait_out(g, k):
    pltpu.make_async_copy(qbufs[k], out_ref(g), osems[k]).wait()

  def transpose_chunk(k):
    src = rbufs[k]
    dst = qbufs[k]

    def e_body(e, carry):
      et = e // 8
      ei = lax.rem(e, 8)
      col = jnp.full((_L,), 0, jnp.int32) + e
      for jb in range(_TC // _L):
        vals = plsc.load_gather(src, [iota + jb * _L, col])
        dst[et, ei, pl.ds(jb * _L, _L)] = vals
      return carry

    lax.fori_loop(0, d, e_body, 0, unroll=2)

  start_gather(0, 0)

  @pl.loop(0, n_s, step=2)
  def _(gg):
    for k in range(2):
      g = gg + k
      nk = 1 - k
      if k == 0:
        @pl.when(gg > 0)
        def _():
          wait_out(g - 1, nk)

        start_gather(g + 1, nk)
      else:
        @pl.when(g + 1 < n_s)
        def _():
          wait_out(g - 1, nk)
          start_gather(g + 1, nk)

      wait_gather(g, k)
      transpose_chunk(k)
      start_out(g, k)

  wait_out(n_s - 2, 0)
  wait_out(n_s - 1, 1)


def kernel(x, table):
  b, n_s = x.shape
  v, d = table.shape
  assert b % _TC == 0 and b // _TC == _NW and d % _L == 0 and n_s % 2 == 0
  scale = math.sqrt(d)
  n_blocks = v // _TC
  tail_n = v - n_blocks * _TC
  assert tail_n % 2 == 0 and v % 2 == 0

  mesh = plsc.VectorSubcoreMesh(core_axis_name="c", subcore_axis_name="s")

  tab_t = jnp.transpose(table)
  tail_flat = table[n_blocks * _TC:, :].reshape(-1)
  relayout = pl.kernel(
      functools.partial(_relayout_body, n_blocks, tail_n, scale),
      mesh=mesh,
      out_type=jax.ShapeDtypeStruct((v // 2, _TC), jnp.float32),
      compiler_params=pltpu.CompilerParams(use_tc_tiling_on_sc=True,
                                           needs_layout_passes=False),
      scratch_types=[
          pltpu.VMEM((2, d, _TC), jnp.float32),
          pltpu.VMEM((2, _TC // 2, _TC), jnp.float32),
          pltpu.VMEM((max(tail_n, 2) * d,), jnp.float32),
          pltpu.SemaphoreType.DMA,
          pltpu.SemaphoreType.DMA,
          pltpu.SemaphoreType.DMA,
          pltpu.SemaphoreType.DMA,
      ],
  )
  tab_rm = relayout(tab_t, tail_flat)

  idx_t = jnp.transpose(x).reshape(n_s, _NW, _TC).astype(jnp.int32)
  tab_lin = tab_rm.reshape(v, d)
  gather = pl.kernel(
      functools.partial(_gather_body, n_s, d),
      mesh=mesh,
      out_type=jax.ShapeDtypeStruct((n_s, d // 8, _NW, 8, _TC), jnp.float32),
      compiler_params=pltpu.CompilerParams(use_tc_tiling_on_sc=False,
                                           needs_layout_passes=False),
      scratch_types=[
          pltpu.VMEM((n_s, _TC), jnp.int32),
          pltpu.VMEM((2, _TC, d), jnp.float32),
          pltpu.VMEM((2, d // 8, 8, _TC), jnp.float32),
          pltpu.SemaphoreType.DMA,
          pltpu.SemaphoreType.DMA,
          pltpu.SemaphoreType.DMA,
          pltpu.SemaphoreType.DMA,
      ],
  )
  q5 = gather(idx_t, tab_lin)

  return jnp.transpose(q5, (2, 4, 0, 1, 3)).reshape(b, n_s, d)


# speedup vs baseline: 1.2084x; 1.2084x over previous
import functools
import math

import jax
import jax.numpy as jnp
from jax import lax
from jax.experimental import pallas as pl
from jax.experimental.pallas import tpu as pltpu
from jax.experimental.pallas import tpu_sc as plsc

_NW = 32
_L = 16
_TC = 128


def _relayout_body(n_blocks, tail_n, scale, tabT_hbm, tail_hbm, out_hbm,
                   ibuf, obuf, tailbuf, gsem0, gsem1, osem0, osem1):
  c = lax.axis_index("c")
  s = lax.axis_index("s")
  wid = s * 2 + c
  d = tabT_hbm.shape[0]
  n_iter = (n_blocks + _NW - 1) // _NW

  ibufs = (ibuf.at[0], ibuf.at[1])
  obufs = (obuf.at[0], obuf.at[1])
  gsems = (gsem0, gsem1)
  osems = (osem0, osem1)

  iota = lax.iota(jnp.int32, _L)
  zeros = jnp.full((_L,), 0, jnp.int32)

  def src_ref(ct):
    return tabT_hbm.at[:, pl.ds(ct * _TC, _TC)]

  def dst_ref(ct):
    return out_hbm.at[pl.ds(ct * (_TC // 2), _TC // 2)]

  def start_read(ct, k):
    pltpu.async_copy(src_ref(ct), ibufs[k], gsems[k])

  def wait_read(ct, k):
    pltpu.make_async_copy(src_ref(ct), ibufs[k], gsems[k]).wait()

  def start_write(ct, k):
    pltpu.async_copy(obufs[k], dst_ref(ct), osems[k])

  def wait_write(ct, k):
    pltpu.make_async_copy(obufs[k], dst_ref(ct), osems[k]).wait()

  c_specs = [(c0, (c0 + iota) // 2, ((c0 + iota) % 2) * d)
             for c0 in range(0, _TC, _L)]

  def transpose_block(k):
    src = ibufs[k]
    dst = obufs[k]

    def e_body(e, carry):
      for c0, i_vec, j_base in c_specs:
        vals = src[e, pl.ds(c0, _L)]
        plsc.store_scatter(dst, [i_vec, j_base + e], vals * scale)
      return carry

    lax.fori_loop(0, d, e_body, 0, unroll=2)

  def ct_of(i):
    return wid + _NW * i

  rd0 = ct_of(0) < n_blocks

  @pl.when(rd0)
  def _():
    start_read(ct_of(0), 0)

  @pl.loop(0, n_iter, step=2)
  def _(ii):
    for kk in range(2):
      i = ii + kk
      nk = 1 - kk
      @pl.when(ct_of(i + 1) < n_blocks)
      def _():
        @pl.when(i >= 1)
        def _():
          wait_write(ct_of(i - 1), nk)

        start_read(ct_of(i + 1), nk)

      @pl.when(ct_of(i) < n_blocks)
      def _():
        wait_read(ct_of(i), kk)
        transpose_block(kk)
        start_write(ct_of(i), kk)

  n_mine = (n_blocks - wid + _NW - 1) // _NW
  for kk in range(2):
    i_k = n_mine - 1 - lax.rem(n_mine - 1 - kk + 2, 2)

    @pl.when(i_k >= 0)
    def _():
      wait_write(ct_of(i_k), kk)

  if tail_n:
    @pl.when(wid == 0)
    def _():
      pltpu.sync_copy(tail_hbm, tailbuf)

      def trow(r, carry):
        for jb in range(2 * d // _L):
          v0 = 2 * r + (jb * _L) // d
          e0 = (jb * _L) % d
          vals = plsc.load_gather(tailbuf, [v0 * d + e0 + iota])
          obufs[0][r, pl.ds(jb * _L, _L)] = vals * scale
        return carry

      lax.fori_loop(0, tail_n // 2, trow, 0)
      pltpu.sync_copy(
          obufs[0].at[pl.ds(0, tail_n // 2)],
          out_hbm.at[pl.ds(n_blocks * (_TC // 2), tail_n // 2)])


def _gather_body(n_s, d, idx_hbm, tab_hbm, out_hbm, idx_v, rbuf, qbuf,
                 gsem0, gsem1, osem0, osem1):
  c = lax.axis_index("c")
  s = lax.axis_index("s")
  wid = s * 2 + c

  pltpu.sync_copy(idx_hbm.at[:, wid], idx_v)

  rbufs = (rbuf.at[0], rbuf.at[1])
  qbufs = (qbuf.at[0], qbuf.at[1])
  gsems = (gsem0, gsem1)
  osems = (osem0, osem1)

  iota = lax.iota(jnp.int32, _L)

  def start_gather(g, k):
    pltpu.async_copy(tab_hbm.at[idx_v.at[g]], rbufs[k], gsems[k])

  def wait_gather(g, k):
    pltpu.make_async_copy(tab_hbm.at[idx_v.at[g]], rbufs[k], gsems[k]).wait()

  def out_ref(g):
    return out_hbm.at[g, :, wid]

  def start_out(g, k):
    pltpu.async_copy(qbufs[k], out_ref(g), osems[k])

  def wait_out(g, k):
    pltpu.make_async_copy(qbufs[k], out_ref(g), osems[k]).wait()

  zeros = jnp.full((_L,), 0, jnp.int32)
  e_specs = [(e0, (e0 + iota) // 8, (e0 + iota) % 8)
             for e0 in range(0, d, _L)]

  def transpose_chunk(k):
    src = rbufs[k]
    dst = qbufs[k]

    def j_body(j, carry):
      j_vec = zeros + j
      for e0, et_vec, ei_vec in e_specs:
        vals = src[j, pl.ds(e0, _L)]
        plsc.store_scatter(dst, [et_vec, ei_vec, j_vec], vals)
      return carry

    lax.fori_loop(0, _TC, j_body, 0, unroll=2)

  start_gather(0, 0)

  @pl.loop(0, n_s, step=2)
  def _(gg):
    for k in range(2):
      g = gg + k
      nk = 1 - k
      if k == 0:
        @pl.when(gg > 0)
        def _():
          wait_out(g - 1, nk)

        start_gather(g + 1, nk)
      else:
        @pl.when(g + 1 < n_s)
        def _():
          wait_out(g - 1, nk)
          start_gather(g + 1, nk)

      wait_gather(g, k)
      transpose_chunk(k)
      start_out(g, k)

  wait_out(n_s - 2, 0)
  wait_out(n_s - 1, 1)


def kernel(x, table):
  b, n_s = x.shape
  v, d = table.shape
  assert b % _TC == 0 and b // _TC == _NW and d % _L == 0 and n_s % 2 == 0
  scale = math.sqrt(d)
  n_blocks = v // _TC
  tail_n = v - n_blocks * _TC
  assert tail_n % 2 == 0 and v % 2 == 0

  mesh = plsc.VectorSubcoreMesh(core_axis_name="c", subcore_axis_name="s")

  tab_t = jnp.transpose(table)
  tail_flat = table[n_blocks * _TC:, :].reshape(-1)
  relayout = pl.kernel(
      functools.partial(_relayout_body, n_blocks, tail_n, scale),
      mesh=mesh,
      out_type=jax.ShapeDtypeStruct((v // 2, _TC), jnp.float32),
      compiler_params=pltpu.CompilerParams(use_tc_tiling_on_sc=True,
                                           needs_layout_passes=False),
      scratch_types=[
          pltpu.VMEM((2, d, _TC), jnp.float32),
          pltpu.VMEM((2, _TC // 2, _TC), jnp.float32),
          pltpu.VMEM((max(tail_n, 2) * d,), jnp.float32),
          pltpu.SemaphoreType.DMA,
          pltpu.SemaphoreType.DMA,
          pltpu.SemaphoreType.DMA,
          pltpu.SemaphoreType.DMA,
      ],
  )
  tab_rm = relayout(tab_t, tail_flat)

  idx_t = jnp.transpose(x).reshape(n_s, _NW, _TC).astype(jnp.int32)
  tab_lin = tab_rm.reshape(v, d)
  gather = pl.kernel(
      functools.partial(_gather_body, n_s, d),
      mesh=mesh,
      out_type=jax.ShapeDtypeStruct((n_s, d // 8, _NW, 8, _TC), jnp.float32),
      compiler_params=pltpu.CompilerParams(use_tc_tiling_on_sc=False,
                                           needs_layout_passes=False),
      scratch_types=[
          pltpu.VMEM((n_s, _TC), jnp.int32),
          pltpu.VMEM((2, _TC, d), jnp.float32),
          pltpu.VMEM((2, d // 8, 8, _TC), jnp.float32),
          pltpu.SemaphoreType.DMA,
          pltpu.SemaphoreType.DMA,
          pltpu.SemaphoreType.DMA,
          pltpu.SemaphoreType.DMA,
      ],
  )
  q5 = gather(idx_t, tab_lin)

  return jnp.transpose(q5, (2, 4, 0, 1, 3)).reshape(b, n_s, d)


# speedup vs baseline: 1.4021x; 1.1602x over previous
import functools
import math

import jax
import jax.numpy as jnp
from jax import lax
from jax.experimental import pallas as pl
from jax.experimental.pallas import tpu as pltpu
from jax.experimental.pallas import tpu_sc as plsc

_NW = 32
_L = 16
_TC = 128
_SB = 256
_CH = 256


def _relayout_body(n_super, tail_n, scale, tabT_hbm, tail_hbm, out_hbm,
                   ibuf, obuf, tailbuf, gsem0, gsem1, osem0, osem1):
  c = lax.axis_index("c")
  s = lax.axis_index("s")
  wid = s * 2 + c
  d = tabT_hbm.shape[0]
  pitch = ibuf.shape[2]
  n_iter = (n_super + _NW - 1) // _NW

  ibufs = (ibuf.at[0], ibuf.at[1])
  obufs = (obuf.at[0], obuf.at[1])
  gsems = (gsem0, gsem1)
  osems = (osem0, osem1)

  iota = lax.iota(jnp.int32, _L)

  def src_ref(sb):
    return tabT_hbm.at[:, pl.ds(sb * _SB, _SB)]

  def dst_ref(sb):
    return out_hbm.at[pl.ds(sb * (_SB // 2), _SB // 2)]

  def start_read(sb, k):
    pltpu.async_copy(src_ref(sb), ibufs[k].at[:, pl.ds(0, _SB)], gsems[k])

  def wait_read(sb, k):
    pltpu.make_async_copy(src_ref(sb), ibufs[k].at[:, pl.ds(0, _SB)],
                          gsems[k]).wait()

  def start_write(sb, k):
    pltpu.async_copy(obufs[k], dst_ref(sb), osems[k])

  def wait_write(sb, k):
    pltpu.make_async_copy(obufs[k], dst_ref(sb), osems[k]).wait()

  def transpose_block(k):
    src = ibufs[k]
    dst = obufs[k]

    def i_body(i, carry):
      for half in range(2):
        col = 2 * i + half
        for e0 in range(0, d, _L):
          vals = plsc.load_gather(src, [e0 + iota, col + (iota - iota)])
          dst[i, pl.ds(half * d + e0, _L)] = vals * scale
      return carry

    lax.fori_loop(0, _SB // 2, i_body, 0, unroll=2)

  def sb_of(i):
    return wid + _NW * i

  @pl.when(sb_of(0) < n_super)
  def _():
    start_read(sb_of(0), 0)

  @pl.loop(0, n_iter, step=2)
  def _(ii):
    for kk in range(2):
      i = ii + kk
      nk = 1 - kk

      @pl.when(sb_of(i + 1) < n_super)
      def _():
        @pl.when(i >= 1)
        def _():
          wait_write(sb_of(i - 1), nk)

        start_read(sb_of(i + 1), nk)

      @pl.when(sb_of(i) < n_super)
      def _():
        wait_read(sb_of(i), kk)
        transpose_block(kk)
        start_write(sb_of(i), kk)

  n_mine = (n_super - wid + _NW - 1) // _NW
  for kk in range(2):
    i_k = n_mine - 1 - lax.rem(n_mine - 1 - kk + 2, 2)

    @pl.when(i_k >= 0)
    def _():
      wait_write(sb_of(i_k), kk)

  if tail_n:
    @pl.when(wid == 0)
    def _():
      pltpu.sync_copy(tail_hbm, tailbuf)

      def trow(r, carry):
        for jb in range(2 * d // _L):
          v0 = 2 * r + (jb * _L) // d
          e0 = (jb * _L) % d
          vals = plsc.load_gather(tailbuf, [v0 * d + e0 + iota])
          obufs[0][r, pl.ds(jb * _L, _L)] = vals * scale
        return carry

      lax.fori_loop(0, tail_n // 2, trow, 0)
      pltpu.sync_copy(
          obufs[0].at[pl.ds(0, tail_n // 2)],
          out_hbm.at[pl.ds(n_super * (_SB // 2), tail_n // 2)])


def _gather_body(n_ch, d, idx_hbm, tab_hbm, out_hbm, idx_v, rbuf, qbuf,
                 gsem0, gsem1, osem0, osem1):
  c = lax.axis_index("c")
  s = lax.axis_index("s")
  wid = s * 2 + c

  pltpu.sync_copy(idx_hbm.at[wid], idx_v)

  rbufs = (rbuf.at[0], rbuf.at[1])
  qbufs = (qbuf.at[0], qbuf.at[1])
  gsems = (gsem0, gsem1)
  osems = (osem0, osem1)

  iota = lax.iota(jnp.int32, _L)
  zeros = jnp.full((_L,), 0, jnp.int32)

  def start_gather(g, k):
    pltpu.async_copy(tab_hbm.at[idx_v.at[g]], rbufs[k], gsems[k])

  def wait_gather(g, k):
    pltpu.make_async_copy(tab_hbm.at[idx_v.at[g]], rbufs[k], gsems[k]).wait()

  def out_ref(g):
    return out_hbm.at[pl.ds(2 * g, 2), :, wid]

  def qview(k):
    return qbufs[k].at[:, :, :, pl.ds(0, _TC)]

  def start_out(g, k):
    pltpu.async_copy(qview(k), out_ref(g), osems[k])

  def wait_out(g, k):
    pltpu.make_async_copy(qview(k), out_ref(g), osems[k]).wait()

  e_specs = [(e0, (e0 + iota) // 8, (e0 + iota) % 8)
             for e0 in range(0, tab_hbm.shape[1], _L)]

  def transpose_chunk(k):
    src = rbufs[k]
    dst = qbufs[k]

    def j_body(j, carry):
      for half in range(2):
        h_vec = zeros + half
        j_vec = zeros + j
        for e0, et_vec, ei_vec in e_specs:
          vals = src[half * _TC + j, pl.ds(e0, _L)]
          plsc.store_scatter(dst, [h_vec, et_vec, ei_vec, j_vec], vals)
      return carry

    lax.fori_loop(0, _TC, j_body, 0, unroll=2)

  start_gather(0, 0)

  @pl.loop(0, n_ch, step=2)
  def _(gg):
    for k in range(2):
      g = gg + k
      nk = 1 - k
      if k == 0:
        @pl.when(gg > 0)
        def _():
          wait_out(g - 1, nk)

        start_gather(g + 1, nk)
      else:
        @pl.when(g + 1 < n_ch)
        def _():
          wait_out(g - 1, nk)
          start_gather(g + 1, nk)

      wait_gather(g, k)
      transpose_chunk(k)
      start_out(g, k)

  wait_out(n_ch - 2, 0)
  wait_out(n_ch - 1, 1)


def kernel(x, table):
  b, n_s = x.shape
  v, d = table.shape
  assert b % _TC == 0 and b // _TC == _NW and d % _L == 0
  assert (2 * n_s) % 4 == 0 and (n_s * _TC) % _CH == 0
  scale = math.sqrt(d)
  n_super = v // _SB
  tail_n = v - n_super * _SB
  assert tail_n % 2 == 0 and v % 2 == 0
  n_ch = n_s * _TC // _CH
  assert n_ch % 2 == 0

  mesh = plsc.VectorSubcoreMesh(core_axis_name="c", subcore_axis_name="s")

  tab_t = jnp.transpose(table)
  tail_flat = table[n_super * _SB:, :].reshape(-1)
  relayout = pl.kernel(
      functools.partial(_relayout_body, n_super, tail_n, scale),
      mesh=mesh,
      out_type=jax.ShapeDtypeStruct((v // 2, _TC), jnp.float32),
      compiler_params=pltpu.CompilerParams(use_tc_tiling_on_sc=True,
                                           needs_layout_passes=False),
      scratch_types=[
          pltpu.VMEM((2, d, _SB + 1), jnp.float32),
          pltpu.VMEM((2, _SB // 2, _TC), jnp.float32),
          pltpu.VMEM((max(tail_n, 2) * d,), jnp.float32),
          pltpu.SemaphoreType.DMA,
          pltpu.SemaphoreType.DMA,
          pltpu.SemaphoreType.DMA,
          pltpu.SemaphoreType.DMA,
      ],
  )
  tab_rm = relayout(tab_t, tail_flat)

  idx3 = (jnp.transpose(x).reshape(n_s, _NW, _TC).transpose(1, 0, 2)
          .reshape(_NW, n_ch, _CH).astype(jnp.int32))
  tab_lin = tab_rm.reshape(v, d)
  gather = pl.kernel(
      functools.partial(_gather_body, n_ch, d),
      mesh=mesh,
      out_type=jax.ShapeDtypeStruct((n_s, d // 8, _NW, 8, _TC), jnp.float32),
      compiler_params=pltpu.CompilerParams(use_tc_tiling_on_sc=False,
                                           needs_layout_passes=False),
      scratch_types=[
          pltpu.VMEM((n_ch, _CH), jnp.int32),
          pltpu.VMEM((2, _CH, d), jnp.float32),
          pltpu.VMEM((2, 2, d // 8, 8, _TC + 1), jnp.float32),
          pltpu.SemaphoreType.DMA,
          pltpu.SemaphoreType.DMA,
          pltpu.SemaphoreType.DMA,
          pltpu.SemaphoreType.DMA,
      ],
  )
  q5 = gather(idx3, tab_lin)

  return jnp.transpose(q5, (2, 4, 0, 1, 3)).reshape(b, n_s, d)


# speedup vs baseline: 1.4068x; 1.0033x over previous
import functools
import math

import jax
import jax.numpy as jnp
from jax import lax
from jax.experimental import pallas as pl
from jax.experimental.pallas import tpu as pltpu
from jax.experimental.pallas import tpu_sc as plsc

_NW = 32
_L = 16
_TC = 128
_SB = 384
_CH = 256


def _relayout_body(n_super, tail_n, scale, tabT_hbm, tail_hbm, out_hbm,
                   ibuf, obuf, tailbuf, gsem0, gsem1, osem0, osem1):
  c = lax.axis_index("c")
  s = lax.axis_index("s")
  wid = s * 2 + c
  d = tabT_hbm.shape[0]
  pitch = ibuf.shape[2]
  n_iter = (n_super + _NW - 1) // _NW

  ibufs = (ibuf.at[0], ibuf.at[1])
  obufs = (obuf.at[0], obuf.at[1])
  gsems = (gsem0, gsem1)
  osems = (osem0, osem1)

  iota = lax.iota(jnp.int32, _L)

  n_rt = d // 8

  def src_ref(sb, rt):
    return tabT_hbm.at[pl.ds(rt * 8, 8), pl.ds(sb * _SB, _SB)]

  def ibuf_ref(k, rt):
    return ibufs[k].at[pl.ds(rt * 8, 8), pl.ds(0, _SB)]

  def dst_ref(sb):
    return out_hbm.at[pl.ds(sb * (_SB // 2), _SB // 2)]

  def start_read(sb, k):
    for rt in range(n_rt):
      pltpu.async_copy(src_ref(sb, rt), ibuf_ref(k, rt), gsems[k])

  def wait_read(sb, k):
    for rt in range(n_rt):
      pltpu.make_async_copy(src_ref(sb, rt), ibuf_ref(k, rt),
                            gsems[k]).wait()

  def start_write(sb, k):
    pltpu.async_copy(obufs[k], dst_ref(sb), osems[k])

  def wait_write(sb, k):
    pltpu.make_async_copy(obufs[k], dst_ref(sb), osems[k]).wait()

  def transpose_block(k):
    src = ibufs[k]
    dst = obufs[k]

    def i_body(i, carry):
      for half in range(2):
        col = 2 * i + half
        for e0 in range(0, d, _L):
          vals = plsc.load_gather(src, [e0 + iota, col + (iota - iota)])
          dst[i, pl.ds(half * d + e0, _L)] = vals * scale
      return carry

    lax.fori_loop(0, _SB // 2, i_body, 0, unroll=2)

  def sb_of(i):
    return wid + _NW * i

  @pl.when(sb_of(0) < n_super)
  def _():
    start_read(sb_of(0), 0)

  @pl.loop(0, n_iter, step=2)
  def _(ii):
    for kk in range(2):
      i = ii + kk
      nk = 1 - kk

      @pl.when(sb_of(i + 1) < n_super)
      def _():
        @pl.when(i >= 1)
        def _():
          wait_write(sb_of(i - 1), nk)

        start_read(sb_of(i + 1), nk)

      @pl.when(sb_of(i) < n_super)
      def _():
        wait_read(sb_of(i), kk)
        transpose_block(kk)
        start_write(sb_of(i), kk)

  n_mine = (n_super - wid + _NW - 1) // _NW
  for kk in range(2):
    i_k = n_mine - 1 - lax.rem(n_mine - 1 - kk + 2, 2)

    @pl.when(i_k >= 0)
    def _():
      wait_write(sb_of(i_k), kk)

  if tail_n:
    @pl.when(wid == 0)
    def _():
      pltpu.sync_copy(tail_hbm, tailbuf)

      def trow(r, carry):
        for jb in range(2 * d // _L):
          v0 = 2 * r + (jb * _L) // d
          e0 = (jb * _L) % d
          vals = plsc.load_gather(tailbuf, [v0 * d + e0 + iota])
          obufs[0][r, pl.ds(jb * _L, _L)] = vals * scale
        return carry

      lax.fori_loop(0, tail_n // 2, trow, 0)
      pltpu.sync_copy(
          obufs[0].at[pl.ds(0, tail_n // 2)],
          out_hbm.at[pl.ds(n_super * (_SB // 2), tail_n // 2)])


def _gather_body(n_ch, d, idx_hbm, tab_hbm, out_hbm, idx_v, rbuf, qbuf,
                 gsem0, gsem1, osem0, osem1):
  c = lax.axis_index("c")
  s = lax.axis_index("s")
  wid = s * 2 + c

  pltpu.sync_copy(idx_hbm.at[wid], idx_v)

  rbufs = (rbuf.at[0], rbuf.at[1])
  qbufs = (qbuf.at[0], qbuf.at[1])
  gsems = (gsem0, gsem1)
  osems = (osem0, osem1)

  iota = lax.iota(jnp.int32, _L)
  zeros = jnp.full((_L,), 0, jnp.int32)

  def start_gather(g, k):
    pltpu.async_copy(tab_hbm.at[idx_v.at[g]], rbufs[k], gsems[k])

  def wait_gather(g, k):
    pltpu.make_async_copy(tab_hbm.at[idx_v.at[g]], rbufs[k], gsems[k]).wait()

  def out_ref(g):
    return out_hbm.at[pl.ds(2 * g, 2), :, wid]

  def qview(k):
    return qbufs[k].at[:, :, :, pl.ds(0, _TC)]

  def start_out(g, k):
    pltpu.async_copy(qview(k), out_ref(g), osems[k])

  def wait_out(g, k):
    pltpu.make_async_copy(qview(k), out_ref(g), osems[k]).wait()

  e_specs = [(e0, (e0 + iota) // 8, (e0 + iota) % 8)
             for e0 in range(0, tab_hbm.shape[1], _L)]

  def transpose_chunk(k):
    src = rbufs[k]
    dst = qbufs[k]

    def j_body(j, carry):
      for half in range(2):
        h_vec = zeros + half
        j_vec = zeros + j
        for e0, et_vec, ei_vec in e_specs:
          vals = src[half * _TC + j, pl.ds(e0, _L)]
          plsc.store_scatter(dst, [h_vec, et_vec, ei_vec, j_vec], vals)
      return carry

    lax.fori_loop(0, _TC, j_body, 0, unroll=2)

  start_gather(0, 0)

  @pl.loop(0, n_ch, step=2)
  def _(gg):
    for k in range(2):
      g = gg + k
      nk = 1 - k
      if k == 0:
        @pl.when(gg > 0)
        def _():
          wait_out(g - 1, nk)

        start_gather(g + 1, nk)
      else:
        @pl.when(g + 1 < n_ch)
        def _():
          wait_out(g - 1, nk)
          start_gather(g + 1, nk)

      wait_gather(g, k)
      transpose_chunk(k)
      start_out(g, k)

  wait_out(n_ch - 2, 0)
  wait_out(n_ch - 1, 1)


def kernel(x, table):
  b, n_s = x.shape
  v, d = table.shape
  assert b % _TC == 0 and b // _TC == _NW and d % _L == 0
  assert (2 * n_s) % 4 == 0 and (n_s * _TC) % _CH == 0
  scale = math.sqrt(d)
  n_super = v // _SB
  tail_n = v - n_super * _SB
  assert tail_n % 2 == 0 and v % 2 == 0
  n_ch = n_s * _TC // _CH
  assert n_ch % 2 == 0

  mesh = plsc.VectorSubcoreMesh(core_axis_name="c", subcore_axis_name="s")

  tab_t = jnp.transpose(table)
  tail_flat = table[n_super * _SB:, :].reshape(-1)
  relayout = pl.kernel(
      functools.partial(_relayout_body, n_super, tail_n, scale),
      mesh=mesh,
      out_type=jax.ShapeDtypeStruct((v // 2, _TC), jnp.float32),
      compiler_params=pltpu.CompilerParams(use_tc_tiling_on_sc=True,
                                           needs_layout_passes=False),
      scratch_types=[
          pltpu.VMEM((2, d, _SB + 1), jnp.float32),
          pltpu.VMEM((2, _SB // 2, _TC), jnp.float32),
          pltpu.VMEM((max(tail_n, 2) * d,), jnp.float32),
          pltpu.SemaphoreType.DMA,
          pltpu.SemaphoreType.DMA,
          pltpu.SemaphoreType.DMA,
          pltpu.SemaphoreType.DMA,
      ],
  )
  tab_rm = relayout(tab_t, tail_flat)

  idx3 = (jnp.transpose(x).reshape(n_s, _NW, _TC).transpose(1, 0, 2)
          .reshape(_NW, n_ch, _CH).astype(jnp.int32))
  tab_lin = tab_rm.reshape(v, d)
  gather = pl.kernel(
      functools.partial(_gather_body, n_ch, d),
      mesh=mesh,
      out_type=jax.ShapeDtypeStruct((n_s, d // 8, _NW, 8, _TC), jnp.float32),
      compiler_params=pltpu.CompilerParams(use_tc_tiling_on_sc=False,
                                           needs_layout_passes=False),
      scratch_types=[
          pltpu.VMEM((n_ch, _CH), jnp.int32),
          pltpu.VMEM((2, _CH, d), jnp.float32),
          pltpu.VMEM((2, 2, d // 8, 8, _TC + 1), jnp.float32),
          pltpu.SemaphoreType.DMA,
          pltpu.SemaphoreType.DMA,
          pltpu.SemaphoreType.DMA,
          pltpu.SemaphoreType.DMA,
      ],
  )
  q5 = gather(idx3, tab_lin)

  return jnp.transpose(q5, (2, 4, 0, 1, 3)).reshape(b, n_s, d)


# speedup vs baseline: 1.6128x; 1.1464x over previous
import functools
import math

import jax
import jax.numpy as jnp
from jax import lax
from jax.experimental import pallas as pl
from jax.experimental.pallas import tpu as pltpu
from jax.experimental.pallas import tpu_sc as plsc

_NW = 32
_L = 16
_TC = 128
_SB = 256
_CH = 256


def _relayout_body(n_super, tail_n, scale, tabT_hbm, tail_hbm, out_hbm,
                   ibuf, obuf, tailbuf, gsem0, gsem1, osem0, osem1):
  c = lax.axis_index("c")
  s = lax.axis_index("s")
  wid = s * 2 + c
  d = tabT_hbm.shape[0]
  pitch = ibuf.shape[2]
  n_iter = (n_super + _NW - 1) // _NW

  ibufs = (ibuf.at[0], ibuf.at[1])
  obufs = (obuf.at[0], obuf.at[1])
  gsems = (gsem0, gsem1)
  osems = (osem0, osem1)

  iota = lax.iota(jnp.int32, _L)

  n_rt = d // 8

  def src_ref(sb, rt):
    return tabT_hbm.at[pl.ds(rt * 8, 8), pl.ds(sb * _SB, _SB)]

  def ibuf_ref(k, rt):
    return ibufs[k].at[pl.ds(rt * 8, 8), pl.ds(0, _SB)]

  def dst_ref(sb):
    return out_hbm.at[pl.ds(sb * (_SB // 2), _SB // 2)]

  def start_read(sb, k):
    for rt in range(n_rt):
      pltpu.async_copy(src_ref(sb, rt), ibuf_ref(k, rt), gsems[k])

  def wait_read(sb, k):
    for rt in range(n_rt):
      pltpu.make_async_copy(src_ref(sb, rt), ibuf_ref(k, rt),
                            gsems[k]).wait()

  def oview(k):
    return obufs[k].at[:, pl.ds(0, _TC)]

  def start_write(sb, k):
    pltpu.async_copy(oview(k), dst_ref(sb), osems[k])

  def wait_write(sb, k):
    pltpu.make_async_copy(oview(k), dst_ref(sb), osems[k]).wait()

  c_specs = [(c0, (c0 + iota) // 2, ((c0 + iota) % 2) * d)
             for c0 in range(0, _SB, _L)]

  def transpose_block(k):
    src = ibufs[k]
    dst = obufs[k]

    def e_body(e, carry):
      for c0, i_vec, j_base in c_specs:
        vals = src[e, pl.ds(c0, _L)]
        plsc.store_scatter(dst, [i_vec, j_base + e], vals * scale)
      return carry

    lax.fori_loop(0, d, e_body, 0, unroll=2)

  def sb_of(i):
    return wid + _NW * i

  @pl.when(sb_of(0) < n_super)
  def _():
    start_read(sb_of(0), 0)

  @pl.loop(0, n_iter, step=2)
  def _(ii):
    for kk in range(2):
      i = ii + kk
      nk = 1 - kk

      @pl.when(sb_of(i + 1) < n_super)
      def _():
        @pl.when(i >= 1)
        def _():
          wait_write(sb_of(i - 1), nk)

        start_read(sb_of(i + 1), nk)

      @pl.when(sb_of(i) < n_super)
      def _():
        wait_read(sb_of(i), kk)
        transpose_block(kk)
        start_write(sb_of(i), kk)

  n_mine = (n_super - wid + _NW - 1) // _NW
  for kk in range(2):
    i_k = n_mine - 1 - lax.rem(n_mine - 1 - kk + 2, 2)

    @pl.when(i_k >= 0)
    def _():
      wait_write(sb_of(i_k), kk)

  if tail_n:
    @pl.when(wid == 0)
    def _():
      pltpu.sync_copy(tail_hbm, tailbuf)

      def trow(r, carry):
        for jb in range(2 * d // _L):
          v0 = 2 * r + (jb * _L) // d
          e0 = (jb * _L) % d
          vals = plsc.load_gather(tailbuf, [v0 * d + e0 + iota])
          obufs[0][r, pl.ds(jb * _L, _L)] = vals * scale
        return carry

      lax.fori_loop(0, tail_n // 2, trow, 0)
      pltpu.sync_copy(
          obufs[0].at[pl.ds(0, tail_n // 2), pl.ds(0, _TC)],
          out_hbm.at[pl.ds(n_super * (_SB // 2), tail_n // 2)])


def _gather_body(n_ch, d, idx_hbm, tab_hbm, out_hbm, idx_v, rbuf, qbuf,
                 gsem0, gsem1, osem0, osem1):
  c = lax.axis_index("c")
  s = lax.axis_index("s")
  wid = s * 2 + c

  pltpu.sync_copy(idx_hbm.at[wid], idx_v)

  rbufs = (rbuf.at[0], rbuf.at[1])
  qbufs = (qbuf.at[0], qbuf.at[1])
  gsems = (gsem0, gsem1)
  osems = (osem0, osem1)

  iota = lax.iota(jnp.int32, _L)
  zeros = jnp.full((_L,), 0, jnp.int32)

  def start_gather(g, k):
    pltpu.async_copy(tab_hbm.at[idx_v.at[g]], rbufs[k], gsems[k])

  def wait_gather(g, k):
    pltpu.make_async_copy(tab_hbm.at[idx_v.at[g]], rbufs[k], gsems[k]).wait()

  def out_ref(g):
    return out_hbm.at[pl.ds(2 * g, 2), :, wid]

  def qview(k):
    return qbufs[k].at[:, :, :, pl.ds(0, _TC)]

  def start_out(g, k):
    pltpu.async_copy(qview(k), out_ref(g), osems[k])

  def wait_out(g, k):
    pltpu.make_async_copy(qview(k), out_ref(g), osems[k]).wait()

  e_specs = [(e0, (e0 + iota) // 8, (e0 + iota) % 8)
             for e0 in range(0, tab_hbm.shape[1], _L)]

  def transpose_chunk(k):
    src = rbufs[k]
    dst = qbufs[k]

    def j_body(j, carry):
      for half in range(2):
        h_vec = zeros + half
        j_vec = zeros + j
        for e0, et_vec, ei_vec in e_specs:
          vals = src[half * _TC + j, pl.ds(e0, _L)]
          plsc.store_scatter(dst, [h_vec, et_vec, ei_vec, j_vec], vals)
      return carry

    lax.fori_loop(0, _TC, j_body, 0, unroll=2)

  start_gather(0, 0)

  @pl.loop(0, n_ch, step=2)
  def _(gg):
    for k in range(2):
      g = gg + k
      nk = 1 - k
      if k == 0:
        @pl.when(gg > 0)
        def _():
          wait_out(g - 1, nk)

        start_gather(g + 1, nk)
      else:
        @pl.when(g + 1 < n_ch)
        def _():
          wait_out(g - 1, nk)
          start_gather(g + 1, nk)

      wait_gather(g, k)
      transpose_chunk(k)
      start_out(g, k)

  wait_out(n_ch - 2, 0)
  wait_out(n_ch - 1, 1)


def kernel(x, table):
  b, n_s = x.shape
  v, d = table.shape
  assert b % _TC == 0 and b // _TC == _NW and d % _L == 0
  assert (2 * n_s) % 4 == 0 and (n_s * _TC) % _CH == 0
  scale = math.sqrt(d)
  n_super = v // _SB
  tail_n = v - n_super * _SB
  assert tail_n % 2 == 0 and v % 2 == 0
  n_ch = n_s * _TC // _CH
  assert n_ch % 2 == 0

  mesh = plsc.VectorSubcoreMesh(core_axis_name="c", subcore_axis_name="s")

  tab_t = jnp.transpose(table)
  tail_flat = table[n_super * _SB:, :].reshape(-1)
  relayout = pl.kernel(
      functools.partial(_relayout_body, n_super, tail_n, scale),
      mesh=mesh,
      out_type=jax.ShapeDtypeStruct((v // 2, _TC), jnp.float32),
      compiler_params=pltpu.CompilerParams(use_tc_tiling_on_sc=True,
                                           needs_layout_passes=False),
      scratch_types=[
          pltpu.VMEM((2, d, _SB + 1), jnp.float32),
          pltpu.VMEM((2, _SB // 2, _TC + 1), jnp.float32),
          pltpu.VMEM((max(tail_n, 2) * d,), jnp.float32),
          pltpu.SemaphoreType.DMA,
          pltpu.SemaphoreType.DMA,
          pltpu.SemaphoreType.DMA,
          pltpu.SemaphoreType.DMA,
      ],
  )
  tab_rm = relayout(tab_t, tail_flat)

  idx3 = (jnp.transpose(x).reshape(n_s, _NW, _TC).transpose(1, 0, 2)
          .reshape(_NW, n_ch, _CH).astype(jnp.int32))
  tab_lin = tab_rm.reshape(v, d)
  gather = pl.kernel(
      functools.partial(_gather_body, n_ch, d),
      mesh=mesh,
      out_type=jax.ShapeDtypeStruct((n_s, d // 8, _NW, 8, _TC), jnp.float32),
      compiler_params=pltpu.CompilerParams(use_tc_tiling_on_sc=False,
                                           needs_layout_passes=False),
      scratch_types=[
          pltpu.VMEM((n_ch, _CH), jnp.int32),
          pltpu.VMEM((2, _CH, d), jnp.float32),
          pltpu.VMEM((2, 2, d // 8, 8, _TC + 1), jnp.float32),
          pltpu.SemaphoreType.DMA,
          pltpu.SemaphoreType.DMA,
          pltpu.SemaphoreType.DMA,
          pltpu.SemaphoreType.DMA,
      ],
  )
  q5 = gather(idx3, tab_lin)

  return jnp.transpose(q5, (2, 4, 0, 1, 3)).reshape(b, n_s, d)


# speedup vs baseline: 2.4650x; 1.5284x over previous
import functools
import math

import jax
import jax.numpy as jnp
from jax import lax
from jax.experimental import pallas as pl
from jax.experimental.pallas import tpu as pltpu
from jax.experimental.pallas import tpu_sc as plsc

_NUM_WORKERS = 32
_CHUNK = 512
_LANES = 16


def _gather_scale_body(n_chunks, d, scale, idx_hbm, table_hbm, out_hbm,
                       idx_v, rows_v, gsem0, gsem1, osem0, osem1):
  c = lax.axis_index("c")
  s = lax.axis_index("s")
  wid = s * 2 + c
  per_w = n_chunks * _CHUNK
  out_base = wid * per_w

  pltpu.sync_copy(idx_hbm.at[wid], idx_v)

  bufs = (rows_v.at[0], rows_v.at[1])
  gsems = (gsem0, gsem1)
  osems = (osem0, osem1)

  def start_gather(g, k):
    pltpu.async_copy(table_hbm.at[idx_v.at[g]], bufs[k], gsems[k])

  def wait_gather(g, k):
    pltpu.make_async_copy(table_hbm.at[idx_v.at[g]], bufs[k], gsems[k]).wait()

  def out_ref(g, k):
    return out_hbm.at[pl.ds(out_base + g * _CHUNK, _CHUNK)]

  def start_out(g, k):
    pltpu.async_copy(bufs[k], out_ref(g, k), osems[k])

  def wait_out(g, k):
    pltpu.make_async_copy(bufs[k], out_ref(g, k), osems[k]).wait()

  def scale_chunk(k):
    buf = bufs[k]

    def scale_body(i, carry):
      for j in range(d // _LANES):
        sl = (i, pl.ds(j * _LANES, _LANES))
        buf[sl] = buf[sl] * scale
      return carry

    lax.fori_loop(0, _CHUNK, scale_body, 0, unroll=8)

  start_gather(0, 0)

  @pl.loop(0, n_chunks, step=2)
  def _(gg):
    for k in range(2):
      g = gg + k
      nk = 1 - k
      if k == 0:
        @pl.when(gg > 0)
        def _():
          wait_out(g - 1, nk)

        start_gather(g + 1, nk)
      else:
        @pl.when(g + 1 < n_chunks)
        def _():
          wait_out(g - 1, nk)
          start_gather(g + 1, nk)

      wait_gather(g, k)
      scale_chunk(k)
      start_out(g, k)

  wait_out(n_chunks - 2, 0)
  wait_out(n_chunks - 1, 1)


def kernel(x, table):
  b, s = x.shape
  v, d = table.shape
  total = b * s
  assert total % (_NUM_WORKERS * _CHUNK) == 0
  assert d % _LANES == 0
  n_chunks = total // (_NUM_WORKERS * _CHUNK)
  assert n_chunks % 2 == 0
  scale = math.sqrt(d)

  idx = x.reshape(_NUM_WORKERS, n_chunks, _CHUNK).astype(jnp.int32)

  mesh = plsc.VectorSubcoreMesh(core_axis_name="c", subcore_axis_name="s")
  body = functools.partial(_gather_scale_body, n_chunks, d, scale)
  out = pl.kernel(
      body,
      mesh=mesh,
      out_type=jax.ShapeDtypeStruct((total, d), jnp.float32),
      compiler_params=pltpu.CompilerParams(use_tc_tiling_on_sc=False),
      scratch_types=[
          pltpu.VMEM((n_chunks, _CHUNK), jnp.int32),
          pltpu.VMEM((2, _CHUNK, d), jnp.float32),
          pltpu.SemaphoreType.DMA,
          pltpu.SemaphoreType.DMA,
          pltpu.SemaphoreType.DMA,
          pltpu.SemaphoreType.DMA,
      ],
  )(idx, table)

  return out.reshape(b, s, d)


# speedup vs baseline: 2.6420x; 1.0718x over previous
import functools
import math

import jax
import jax.numpy as jnp
from jax import lax
from jax.experimental import pallas as pl
from jax.experimental.pallas import tpu as pltpu
from jax.experimental.pallas import tpu_sc as plsc

_NW = 32
_L = 16
_TC = 128
_CH = 256


def _gather_body(n_ch, d, scale, idx_hbm, tab_hbm, out_hbm, idx_v, rbuf,
                 qbuf, gsem0, gsem1, osem0, osem1):
  c = lax.axis_index("c")
  s = lax.axis_index("s")
  wid = s * 2 + c

  pltpu.sync_copy(idx_hbm.at[wid], idx_v)

  rbufs = (rbuf.at[0], rbuf.at[1])
  qbufs = (qbuf.at[0], qbuf.at[1])
  gsems = (gsem0, gsem1)
  osems = (osem0, osem1)

  iota = lax.iota(jnp.int32, _L)
  zeros = jnp.full((_L,), 0, jnp.int32)

  def start_gather(g, k):
    pltpu.async_copy(tab_hbm.at[idx_v.at[g]], rbufs[k], gsems[k])

  def wait_gather(g, k):
    pltpu.make_async_copy(tab_hbm.at[idx_v.at[g]], rbufs[k], gsems[k]).wait()

  def out_ref(g):
    return out_hbm.at[pl.ds(2 * g, 2), :, wid]

  def qview(k):
    return qbufs[k].at[:, :, :, pl.ds(0, _TC)]

  def start_out(g, k):
    pltpu.async_copy(qview(k), out_ref(g), osems[k])

  def wait_out(g, k):
    pltpu.make_async_copy(qview(k), out_ref(g), osems[k]).wait()

  e_specs = [(e0, (e0 + iota) // 8, (e0 + iota) % 8)
             for e0 in range(0, d, _L)]

  def transpose_chunk(k):
    src = rbufs[k]
    dst = qbufs[k]

    def j_body(j, carry):
      for half in range(2):
        h_vec = zeros + half
        j_vec = zeros + j
        for e0, et_vec, ei_vec in e_specs:
          vals = src[half * _TC + j, pl.ds(e0, _L)]
          plsc.store_scatter(dst, [h_vec, et_vec, ei_vec, j_vec],
                             vals * scale)
      return carry

    lax.fori_loop(0, _TC, j_body, 0, unroll=2)

  start_gather(0, 0)

  @pl.loop(0, n_ch, step=2)
  def _(gg):
    for k in range(2):
      g = gg + k
      nk = 1 - k
      if k == 0:
        @pl.when(gg > 0)
        def _():
          wait_out(g - 1, nk)

        start_gather(g + 1, nk)
      else:
        @pl.when(g + 1 < n_ch)
        def _():
          wait_out(g - 1, nk)
          start_gather(g + 1, nk)

      wait_gather(g, k)
      transpose_chunk(k)
      start_out(g, k)

  wait_out(n_ch - 2, 0)
  wait_out(n_ch - 1, 1)


def kernel(x, table):
  b, n_s = x.shape
  v, d = table.shape
  assert b % _TC == 0 and b // _TC == _NW and d % _L == 0
  assert (n_s * _TC) % _CH == 0
  scale = math.sqrt(d)
  n_ch = n_s * _TC // _CH
  assert n_ch % 2 == 0

  mesh = plsc.VectorSubcoreMesh(core_axis_name="c", subcore_axis_name="s")

  idx3 = (jnp.transpose(x).reshape(n_s, _NW, _TC).transpose(1, 0, 2)
          .reshape(_NW, n_ch, _CH).astype(jnp.int32))
  gather = pl.kernel(
      functools.partial(_gather_body, n_ch, d, scale),
      mesh=mesh,
      out_type=jax.ShapeDtypeStruct((n_s, d // 8, _NW, 8, _TC), jnp.float32),
      compiler_params=pltpu.CompilerParams(use_tc_tiling_on_sc=False,
                                           needs_layout_passes=False),
      scratch_types=[
          pltpu.VMEM((n_ch, _CH), jnp.int32),
          pltpu.VMEM((2, _CH, d), jnp.float32),
          pltpu.VMEM((2, 2, d // 8, 8, _TC + 1), jnp.float32),
          pltpu.SemaphoreType.DMA,
          pltpu.SemaphoreType.DMA,
          pltpu.SemaphoreType.DMA,
          pltpu.SemaphoreType.DMA,
      ],
  )
  q5 = gather(idx3, table)

  return jnp.transpose(q5, (2, 4, 0, 1, 3)).reshape(b, n_s, d)
